# baseline (speedup 1.0000x reference)
import functools

import jax
import jax.numpy as jnp
from jax import lax
from jax.experimental import pallas as pl
from jax.experimental.pallas import tpu as pltpu
from jax.experimental.pallas import tpu_sc as plsc

A = 8192
P = 4096
K = 16
NPART = 16
L = 16
NC = 2
NS = 16
NW = NC * NS
APW = A // NW
BIG = 1e30


def _sc_topk(ax, ay, av, ap, px, py, lo_t, hi_t):
    mesh = plsc.VectorSubcoreMesh(
        core_axis_name="c", subcore_axis_name="s",
        num_cores=NC, num_subcores=NS)

    @functools.partial(
        pl.kernel,
        out_type=(jax.ShapeDtypeStruct((A * K,), jnp.int32),
                  jax.ShapeDtypeStruct((A * K,), jnp.int32)),
        mesh=mesh,
        compiler_params=pltpu.CompilerParams(needs_layout_passes=False),
        scratch_types=[
            pltpu.VMEM((P,), jnp.float32),
            pltpu.VMEM((P,), jnp.float32),
            pltpu.VMEM((APW,), jnp.float32),
            pltpu.VMEM((APW,), jnp.float32),
            pltpu.VMEM((APW,), jnp.int32),
            pltpu.VMEM((APW,), jnp.int32),
            pltpu.VMEM((NPART,), jnp.int32),
            pltpu.VMEM((NPART,), jnp.int32),
            pltpu.VMEM((APW * K,), jnp.int32),
            pltpu.VMEM((APW * K,), jnp.int32),
        ],
    )
    def k(ax_h, ay_h, av_h, ap_h, px_h, py_h, lo_h, hi_h,
          out0_h, out1_h,
          px_v, py_v, ax_v, ay_v, av_v, ap_v, lo_v, hi_v, o0_v, o1_v):
        wid = lax.axis_index("s") * NC + lax.axis_index("c")
        base = wid * APW
        pltpu.sync_copy(px_h, px_v)
        pltpu.sync_copy(py_h, py_v)
        pltpu.sync_copy(lo_h, lo_v)
        pltpu.sync_copy(hi_h, hi_v)
        pltpu.sync_copy(ax_h.at[pl.ds(base, APW)], ax_v)
        pltpu.sync_copy(ay_h.at[pl.ds(base, APW)], ay_v)
        pltpu.sync_copy(av_h.at[pl.ds(base, APW)], av_v)
        pltpu.sync_copy(ap_h.at[pl.ds(base, APW)], ap_v)

        iota = lax.iota(jnp.int32, L)
        lo_r = lo_v[...]
        hi_r = hi_v[...]

        def _splat(vec, lane_v):
            return jnp.take_along_axis(vec, lane_v, axis=0,
                                       mode="promise_in_bounds")

        def agent_body(a, _):
            grp = (a // L) * L
            lane_v = jnp.full((L,), a - grp, jnp.int32)
            ax_b = _splat(ax_v[pl.ds(grp, L)], lane_v)
            ay_b = _splat(ay_v[pl.ds(grp, L)], lane_v)
            pt_b = _splat(ap_v[pl.ds(grp, L)], lane_v)
            vl_b = _splat(av_v[pl.ds(grp, L)], lane_v)
            ok = vl_b > 0
            lo_b = jnp.where(ok, _splat(lo_r, pt_b), 0)
            hi_b = jnp.where(ok, _splat(hi_r, pt_b), 0)
            lo_s = lo_b[0]
            hi_s = hi_b[0]
            c0 = (lo_s // L) * L
            nch = (hi_s - c0 + (L - 1)) // L

            def chunk(c, carry):
                bd, bi = carry
                j0 = c0 + c * L
                jv = jnp.full((L,), j0, jnp.int32) + iota
                dx = px_v[pl.ds(j0, L)] - ax_b
                dy = py_v[pl.ds(j0, L)] - ay_b
                d2 = dx * dx + dy * dy
                m = (jv >= lo_b) & (jv < hi_b)
                d2 = jnp.where(m, d2, BIG)

                def merge(args):
                    bd, bi, d2, jv = args
                    sd, si = plsc.sort_key_val(d2, jv)
                    rd = lax.rev(sd, (0,))
                    ri = lax.rev(si, (0,))
                    keep = (bd < rd) | ((bd == rd) & (bi <= ri))
                    nd = jnp.where(keep, bd, rd)
                    ni = jnp.where(keep, bi, ri)
                    md, mi = plsc.sort_key_val(nd, ni)
                    return (md, mi)

                return lax.cond(jnp.min(d2) < bd[L - 1], merge,
                                lambda args: (args[0], args[1]),
                                (bd, bi, d2, jv))

            bd0 = jnp.full((L,), BIG, jnp.float32)
            bi0 = jnp.zeros((L,), jnp.int32)
            _, bi = lax.fori_loop(0, nch, chunk, (bd0, bi0))

            cnt_b = hi_b - lo_b
            fm = iota - cnt_b
            fi = jnp.where(fm < lo_b, fm, hi_b + (fm - lo_b))
            outi = jnp.where(iota < cnt_b, bi, fi)
            o0_v[pl.ds(a * K, K)] = outi
            o1_v[pl.ds(a * K, K)] = jnp.full((L,), base + a, jnp.int32)
            return 0

        lax.fori_loop(0, APW, agent_body, 0)
        pltpu.sync_copy(o0_v, out0_h.at[pl.ds(base * K, APW * K)])
        pltpu.sync_copy(o1_v, out1_h.at[pl.ds(base * K, APW * K)])

    return k(ax, ay, av, ap, px, py, lo_t, hi_t)


def kernel(agent_position, agent_valid, agent_partition,
           polyline_start_position, polyline_partition):
    ax = agent_position[:, 0].astype(jnp.float32)
    ay = agent_position[:, 1].astype(jnp.float32)
    av = agent_valid.astype(jnp.int32)
    ap = agent_partition.astype(jnp.int32)
    px = polyline_start_position[:, 0].astype(jnp.float32)
    py = polyline_start_position[:, 1].astype(jnp.float32)
    ids = jnp.arange(NPART, dtype=polyline_partition.dtype)
    lo_t = jnp.searchsorted(polyline_partition, ids, side="left").astype(jnp.int32)
    hi_t = jnp.searchsorted(polyline_partition, ids, side="right").astype(jnp.int32)
    row0, row1 = _sc_topk(ax, ay, av, ap, px, py, lo_t, hi_t)
    return jnp.stack([row0, row1], axis=0)


# speedup vs baseline: 1.7050x; 1.7050x over previous
import functools

import jax
import jax.numpy as jnp
from jax import lax
from jax.experimental import pallas as pl
from jax.experimental.pallas import tpu as pltpu
from jax.experimental.pallas import tpu_sc as plsc

A = 8192
P = 4096
K = 16
NPART = 16
L = 16
NC = 2
NS = 16
NW = NC * NS
APW = A // NW
BIG = 1e30


def _sc_topk(ax, ay, av, ap, px, py, lo_t, hi_t):
    mesh = plsc.VectorSubcoreMesh(
        core_axis_name="c", subcore_axis_name="s",
        num_cores=NC, num_subcores=NS)

    @functools.partial(
        pl.kernel,
        out_type=(jax.ShapeDtypeStruct((A * K,), jnp.int32),
                  jax.ShapeDtypeStruct((A * K,), jnp.int32)),
        mesh=mesh,
        compiler_params=pltpu.CompilerParams(needs_layout_passes=False),
        scratch_types=[
            pltpu.VMEM((P + 2 * L,), jnp.float32),
            pltpu.VMEM((P + 2 * L,), jnp.float32),
            pltpu.VMEM((APW,), jnp.float32),
            pltpu.VMEM((APW,), jnp.float32),
            pltpu.VMEM((APW,), jnp.int32),
            pltpu.VMEM((APW,), jnp.int32),
            pltpu.VMEM((NPART,), jnp.int32),
            pltpu.VMEM((NPART,), jnp.int32),
            pltpu.VMEM((APW * K,), jnp.int32),
            pltpu.VMEM((APW * K,), jnp.int32),
        ],
    )
    def k(ax_h, ay_h, av_h, ap_h, px_h, py_h, lo_h, hi_h,
          out0_h, out1_h,
          px_v, py_v, ax_v, ay_v, av_v, ap_v, lo_v, hi_v, o0_v, o1_v):
        wid = lax.axis_index("s") * NC + lax.axis_index("c")
        base = wid * APW
        pltpu.sync_copy(px_h, px_v)
        pltpu.sync_copy(py_h, py_v)
        pltpu.sync_copy(lo_h, lo_v)
        pltpu.sync_copy(hi_h, hi_v)
        pltpu.sync_copy(ax_h.at[pl.ds(base, APW)], ax_v)
        pltpu.sync_copy(ay_h.at[pl.ds(base, APW)], ay_v)
        pltpu.sync_copy(av_h.at[pl.ds(base, APW)], av_v)
        pltpu.sync_copy(ap_h.at[pl.ds(base, APW)], ap_v)

        iota = lax.iota(jnp.int32, L)
        lo_r = lo_v[...]
        hi_r = hi_v[...]

        def _splat(vec, lane_v):
            return jnp.take_along_axis(vec, lane_v, axis=0,
                                       mode="promise_in_bounds")

        def agent_body(a, _):
            grp = (a // L) * L
            lane_v = jnp.full((L,), a - grp, jnp.int32)
            ax_b = _splat(ax_v[pl.ds(grp, L)], lane_v)
            ay_b = _splat(ay_v[pl.ds(grp, L)], lane_v)
            pt_b = _splat(ap_v[pl.ds(grp, L)], lane_v)
            vl_b = _splat(av_v[pl.ds(grp, L)], lane_v)
            ok = vl_b > 0
            lo_b = jnp.where(ok, _splat(lo_r, pt_b), 0)
            hi_b = jnp.where(ok, _splat(hi_r, pt_b), 0)
            lo_s = lo_b[0]
            hi_s = hi_b[0]
            c0 = (lo_s // L) * L
            nch = (hi_s - c0 + (L - 1)) // L
            nhalf = (nch + 1) // 2

            def one_chunk(j0, bd, bi):
                jv = jnp.full((L,), j0, jnp.int32) + iota
                dx = px_v[pl.ds(j0, L)] - ax_b
                dy = py_v[pl.ds(j0, L)] - ay_b
                d2 = dx * dx + dy * dy
                m = (jv >= lo_b) & (jv < hi_b)
                d2 = jnp.where(m, d2, BIG)
                sd, si = plsc.sort_key_val(d2, jv)
                rd = lax.rev(sd, (0,))
                ri = lax.rev(si, (0,))
                keep = (bd < rd) | ((bd == rd) & (bi <= ri))
                nd = jnp.where(keep, bd, rd)
                ni = jnp.where(keep, bi, ri)
                md, mi = plsc.sort_key_val(nd, ni)
                return md, mi

            def chunk(t, carry):
                bdA, biA, bdB, biB = carry
                bdA, biA = one_chunk(c0 + t * L, bdA, biA)
                bdB, biB = one_chunk(c0 + (nhalf + t) * L, bdB, biB)
                return (bdA, biA, bdB, biB)

            bd0 = jnp.full((L,), BIG, jnp.float32)
            bi0 = jnp.zeros((L,), jnp.int32)
            bdA, biA, bdB, biB = lax.fori_loop(
                0, nhalf, chunk, (bd0, bi0, bd0, bi0))
            rd = lax.rev(bdB, (0,))
            ri = lax.rev(biB, (0,))
            keep = (bdA < rd) | ((bdA == rd) & (biA <= ri))
            nd = jnp.where(keep, bdA, rd)
            ni = jnp.where(keep, biA, ri)
            _, bi = plsc.sort_key_val(nd, ni)

            cnt_b = hi_b - lo_b
            fm = iota - cnt_b
            fi = jnp.where(fm < lo_b, fm, hi_b + (fm - lo_b))
            outi = jnp.where(iota < cnt_b, bi, fi)
            o0_v[pl.ds(a * K, K)] = outi
            o1_v[pl.ds(a * K, K)] = jnp.full((L,), base + a, jnp.int32)
            return 0

        lax.fori_loop(0, APW, agent_body, 0)
        pltpu.sync_copy(o0_v, out0_h.at[pl.ds(base * K, APW * K)])
        pltpu.sync_copy(o1_v, out1_h.at[pl.ds(base * K, APW * K)])

    return k(ax, ay, av, ap, px, py, lo_t, hi_t)


def kernel(agent_position, agent_valid, agent_partition,
           polyline_start_position, polyline_partition):
    ax = agent_position[:, 0].astype(jnp.float32)
    ay = agent_position[:, 1].astype(jnp.float32)
    av = agent_valid.astype(jnp.int32)
    ap = agent_partition.astype(jnp.int32)
    px = jnp.pad(polyline_start_position[:, 0].astype(jnp.float32), (0, 2 * L))
    py = jnp.pad(polyline_start_position[:, 1].astype(jnp.float32), (0, 2 * L))
    ids = jnp.arange(NPART, dtype=polyline_partition.dtype)
    lo_t = jnp.searchsorted(polyline_partition, ids, side="left").astype(jnp.int32)
    hi_t = jnp.searchsorted(polyline_partition, ids, side="right").astype(jnp.int32)
    row0, row1 = _sc_topk(ax, ay, av, ap, px, py, lo_t, hi_t)
    return jnp.stack([row0, row1], axis=0)


# speedup vs baseline: 1.7836x; 1.0462x over previous
import functools

import jax
import jax.numpy as jnp
from jax import lax
from jax.experimental import pallas as pl
from jax.experimental.pallas import tpu as pltpu
from jax.experimental.pallas import tpu_sc as plsc

A = 8192
P = 4096
K = 16
NPART = 16
L = 16
NC = 2
NS = 16
NW = NC * NS
APW = A // NW
BIG = 1e30


def _sc_topk(ax, ay, av, ap, px, py, lo_t, hi_t):
    mesh = plsc.VectorSubcoreMesh(
        core_axis_name="c", subcore_axis_name="s",
        num_cores=NC, num_subcores=NS)

    @functools.partial(
        pl.kernel,
        out_type=(jax.ShapeDtypeStruct((A * K,), jnp.int32),
                  jax.ShapeDtypeStruct((A * K,), jnp.int32)),
        mesh=mesh,
        compiler_params=pltpu.CompilerParams(needs_layout_passes=False),
        scratch_types=[
            pltpu.VMEM((P + 2 * L,), jnp.float32),
            pltpu.VMEM((P + 2 * L,), jnp.float32),
            pltpu.VMEM((APW,), jnp.float32),
            pltpu.VMEM((APW,), jnp.float32),
            pltpu.VMEM((APW,), jnp.int32),
            pltpu.VMEM((APW,), jnp.int32),
            pltpu.VMEM((NPART,), jnp.int32),
            pltpu.VMEM((NPART,), jnp.int32),
            pltpu.VMEM((APW * K,), jnp.int32),
            pltpu.VMEM((APW * K,), jnp.int32),
        ],
    )
    def k(ax_h, ay_h, av_h, ap_h, px_h, py_h, lo_h, hi_h,
          out0_h, out1_h,
          px_v, py_v, ax_v, ay_v, av_v, ap_v, lo_v, hi_v, o0_v, o1_v):
        wid = lax.axis_index("s") * NC + lax.axis_index("c")
        base = wid * APW
        pltpu.sync_copy(px_h, px_v)
        pltpu.sync_copy(py_h, py_v)
        pltpu.sync_copy(lo_h, lo_v)
        pltpu.sync_copy(hi_h, hi_v)
        pltpu.sync_copy(ax_h.at[pl.ds(base, APW)], ax_v)
        pltpu.sync_copy(ay_h.at[pl.ds(base, APW)], ay_v)
        pltpu.sync_copy(av_h.at[pl.ds(base, APW)], av_v)
        pltpu.sync_copy(ap_h.at[pl.ds(base, APW)], ap_v)

        iota = lax.iota(jnp.int32, L)
        lo_r = lo_v[...]
        hi_r = hi_v[...]

        def _splat(vec, lane_v):
            return jnp.take_along_axis(vec, lane_v, axis=0,
                                       mode="promise_in_bounds")

        def agent_body(a, _):
            grp = (a // L) * L
            lane_v = jnp.full((L,), a - grp, jnp.int32)
            ax_b = _splat(ax_v[pl.ds(grp, L)], lane_v)
            ay_b = _splat(ay_v[pl.ds(grp, L)], lane_v)
            pt_b = _splat(ap_v[pl.ds(grp, L)], lane_v)
            vl_b = _splat(av_v[pl.ds(grp, L)], lane_v)
            ok = vl_b > 0
            lo_b = jnp.where(ok, _splat(lo_r, pt_b), 0)
            hi_b = jnp.where(ok, _splat(hi_r, pt_b), 0)
            c0_b = lo_b & jnp.int32(-L)
            nch_b = lax.shift_right_logical(hi_b - c0_b + (L - 1), 4)
            nh_b = lax.shift_right_logical(nch_b + 1, 1)
            packed = jnp.left_shift(nh_b, 16) | c0_b
            pk = packed[0]
            c0 = pk & jnp.int32(0xFFFF)
            nhalf = lax.shift_right_logical(pk, 16)

            def one_chunk(j0, bd, bi):
                jv = jnp.full((L,), j0, jnp.int32) + iota
                dx = px_v[pl.ds(j0, L)] - ax_b
                dy = py_v[pl.ds(j0, L)] - ay_b
                d2 = dx * dx + dy * dy
                m = (jv >= lo_b) & (jv < hi_b)
                d2 = jnp.where(m, d2, BIG)
                sd, si = plsc.sort_key_val(d2, jv)
                rd = lax.rev(sd, (0,))
                ri = lax.rev(si, (0,))
                keep = (bd < rd) | ((bd == rd) & (bi <= ri))
                nd = jnp.where(keep, bd, rd)
                ni = jnp.where(keep, bi, ri)
                md, mi = plsc.sort_key_val(nd, ni)
                return md, mi

            def chunk(t, carry):
                bdA, biA, bdB, biB = carry
                bdA, biA = one_chunk(c0 + t * L, bdA, biA)
                bdB, biB = one_chunk(c0 + (nhalf + t) * L, bdB, biB)
                return (bdA, biA, bdB, biB)

            bd0 = jnp.full((L,), BIG, jnp.float32)
            bi0 = jnp.zeros((L,), jnp.int32)
            bdA, biA, bdB, biB = lax.fori_loop(
                0, nhalf, chunk, (bd0, bi0, bd0, bi0))
            rd = lax.rev(bdB, (0,))
            ri = lax.rev(biB, (0,))
            keep = (bdA < rd) | ((bdA == rd) & (biA <= ri))
            nd = jnp.where(keep, bdA, rd)
            ni = jnp.where(keep, biA, ri)
            _, bi = plsc.sort_key_val(nd, ni)

            cnt_b = hi_b - lo_b
            fm = iota - cnt_b
            fi = jnp.where(fm < lo_b, fm, hi_b + (fm - lo_b))
            outi = jnp.where(iota < cnt_b, bi, fi)
            o0_v[pl.ds(a * K, K)] = outi
            o1_v[pl.ds(a * K, K)] = jnp.full((L,), base + a, jnp.int32)
            return 0

        lax.fori_loop(0, APW, agent_body, 0)
        pltpu.sync_copy(o0_v, out0_h.at[pl.ds(base * K, APW * K)])
        pltpu.sync_copy(o1_v, out1_h.at[pl.ds(base * K, APW * K)])

    return k(ax, ay, av, ap, px, py, lo_t, hi_t)


def kernel(agent_position, agent_valid, agent_partition,
           polyline_start_position, polyline_partition):
    ax = agent_position[:, 0].astype(jnp.float32)
    ay = agent_position[:, 1].astype(jnp.float32)
    av = agent_valid.astype(jnp.int32)
    ap = agent_partition.astype(jnp.int32)
    px = jnp.pad(polyline_start_position[:, 0].astype(jnp.float32), (0, 2 * L))
    py = jnp.pad(polyline_start_position[:, 1].astype(jnp.float32), (0, 2 * L))
    ids = jnp.arange(NPART, dtype=polyline_partition.dtype)
    lo_t = jnp.searchsorted(polyline_partition, ids, side="left").astype(jnp.int32)
    hi_t = jnp.searchsorted(polyline_partition, ids, side="right").astype(jnp.int32)
    row0, row1 = _sc_topk(ax, ay, av, ap, px, py, lo_t, hi_t)
    return jnp.stack([row0, row1], axis=0)
